# Initial kernel scaffold; baseline (speedup 1.0000x reference)
import jax
import jax.numpy as jnp
from jax.experimental import pallas as pl


def kernel(attention_scores, uniform):
    raise NotImplementedError("write your pallas kernel here")


# speedup vs baseline: 1.6665x; 1.6665x over previous
import jax
import jax.numpy as jnp
from jax.experimental import pallas as pl
from jax.experimental.pallas import tpu as pltpu

_TEMPERATURE = 1.0
_EPS = 1e-08
_B, _N = 1024, 4096
_ROWS = 128


def _body(x_ref, u_ref, o_ref):
    x = x_ref[...]
    u = u_ref[...]
    probs = jax.nn.sigmoid(x)
    gumbel = -jnp.log(-jnp.log(u + _EPS) + _EPS)
    logits = jnp.log(probs + _EPS) - jnp.log(1.0 - probs + _EPS) + gumbel
    logits = logits / _TEMPERATURE
    m = jnp.max(logits, axis=-1, keepdims=True)
    e = jnp.exp(logits - m)
    s = jnp.sum(e, axis=-1, keepdims=True)
    y = e / s
    y_hard = (y > 0.5).astype(jnp.float32)
    o_ref[...] = (y_hard - y) + y


def kernel(attention_scores, uniform):
    grid = (_B // _ROWS,)
    spec = pl.BlockSpec((_ROWS, _N), lambda i: (i, 0))
    return pl.pallas_call(
        _body,
        grid=grid,
        in_specs=[spec, spec],
        out_specs=spec,
        out_shape=jax.ShapeDtypeStruct((_B, _N), jnp.float32),
        compiler_params=pltpu.CompilerParams(
            dimension_semantics=("arbitrary",),
        ),
    )(attention_scores, uniform)
